# baseline (speedup 1.0000x reference)
import functools

import jax
import jax.numpy as jnp
from jax import lax
from jax.experimental import pallas as pl
from jax.experimental.pallas import tpu as pltpu
from jax.experimental.pallas import tpu_sc as plsc

VOCAB = 100000
HIDDEN = 768
B, L = 1024, 200
N = B * L

_INFO = plsc.get_sparse_core_info()
NC, NS, LANES = _INFO.num_cores, _INFO.num_subcores, _INFO.num_lanes
NW = NC * NS
PER_W = N // NW
CHUNK = 40
NCHUNKS = PER_W // CHUNK
CBLKS = HIDDEN // LANES


def _body(token_hbm, segment_hbm, token_tab, pos_tab, seg_tab, out_hbm,
          tok_idx, seg_idx, buf_a, buf_b, buf_c, sem):
    wid = lax.axis_index("s") * NC + lax.axis_index("c")
    base = wid * PER_W

    def chunk_step(c, _):
        row0 = base + c * CHUNK
        pltpu.sync_copy(token_hbm.at[pl.ds(row0, CHUNK)], tok_idx)
        pltpu.sync_copy(segment_hbm.at[pl.ds(row0, CHUNK)], seg_idx)
        cp_a = pltpu.async_copy(token_tab.at[tok_idx], buf_a, sem)
        cp_b = pltpu.async_copy(pos_tab.at[tok_idx], buf_b, sem)
        cp_c = pltpu.async_copy(seg_tab.at[seg_idx], buf_c, sem)
        cp_a.wait()
        cp_b.wait()
        cp_c.wait()

        def row_step(r, _):
            for j in range(CBLKS):
                sl = pl.ds(j * LANES, LANES)
                buf_a[r, sl] = buf_a[r, sl] + buf_b[r, sl] + buf_c[r, sl]
            return 0

        lax.fori_loop(0, CHUNK, row_step, 0)
        pltpu.sync_copy(buf_a, out_hbm.at[pl.ds(row0, CHUNK)])
        return 0

    lax.fori_loop(0, NCHUNKS, chunk_step, 0)


@jax.jit
def _run(token_flat, segment_flat, token_table, pos_table, seg_table):
    mesh = plsc.VectorSubcoreMesh(core_axis_name="c", subcore_axis_name="s")
    kern = pl.kernel(
        _body,
        out_type=jax.ShapeDtypeStruct((N, HIDDEN), jnp.float32),
        mesh=mesh,
        scratch_types=[
            pltpu.VMEM((CHUNK,), jnp.int32),
            pltpu.VMEM((CHUNK,), jnp.int32),
            pltpu.VMEM((CHUNK, HIDDEN), jnp.float32),
            pltpu.VMEM((CHUNK, HIDDEN), jnp.float32),
            pltpu.VMEM((CHUNK, HIDDEN), jnp.float32),
            pltpu.SemaphoreType.DMA,
        ],
    )
    return kern(token_flat, segment_flat, token_table, pos_table, seg_table)


def kernel(token, segment, token_table, pos_table, seg_table):
    token_flat = token.reshape(N).astype(jnp.int32)
    segment_flat = segment.reshape(N).astype(jnp.int32)
    out = _run(token_flat, segment_flat, token_table, pos_table, seg_table)
    return out.reshape(B, L, HIDDEN)
